# Initial kernel scaffold; baseline (speedup 1.0000x reference)
import functools

import jax
import jax.numpy as jnp
from jax import lax
from jax.experimental import pallas as pl
from jax.experimental.pallas import tpu as pltpu
from jax.experimental.pallas import tpu_sc as plsc

_NS = 16
_NC = 2
_LANES = 16
_BLK = 128
_IDXROWS = 8
_ROWCH = 128


def _ceil_to(x, m):
    return (x + m - 1) // m * m


def _gnn_body(npad, ept_rows, nfin, dh,
              feat2, src2d, dst2d, out2,
              acc, deg, sidx, didx, rows, ones_b, fbuf, abuf, dbuf):
    c = lax.axis_index("c")
    s = lax.axis_index("s")
    coff = c * npad

    one_v = jnp.full((_LANES,), 1.0, jnp.float32)
    zero_v = jnp.zeros((_LANES,), jnp.float32)

    @pl.loop(0, _BLK)
    def _(i):
        ones_b[i, :] = one_v
        dbuf[i, :] = zero_v
        for q in range(dh // _LANES):
            fbuf[i, pl.ds(_LANES * q, _LANES)] = zero_v

    @pl.loop(0, nfin)
    def _(k):
        r0 = (s * nfin + k) * _ROWCH
        pltpu.sync_copy(fbuf, acc.at[pl.ds(r0, _ROWCH)])
        pltpu.sync_copy(dbuf, deg.at[pl.ds(r0, _ROWCH)])

    plsc.subcore_barrier()

    ebase = s * ept_rows

    @pl.loop(0, ept_rows // _IDXROWS)
    def _(jb):
        rb = ebase + jb * _IDXROWS
        pltpu.sync_copy(src2d.at[pl.ds(rb, _IDXROWS)], sidx)
        pltpu.sync_copy(dst2d.at[pl.ds(rb, _IDXROWS)], didx)
        for r in range(_IDXROWS):
            for q in range(_BLK // _LANES):
                sl = pl.ds(_LANES * q, _LANES)
                sidx[r, sl] = sidx[r, sl] + coff
        for j in range(_IDXROWS):
            pltpu.sync_copy(feat2.at[sidx.at[j]], rows)
            pltpu.sync_copy(rows, acc.at[didx.at[j]], add=True)
            pltpu.sync_copy(ones_b, deg.at[didx.at[j]], add=True)

    plsc.subcore_barrier()

    @pl.loop(0, nfin)
    def _(k):
        r0 = (s * nfin + k) * _ROWCH
        pltpu.sync_copy(feat2.at[pl.ds(coff + r0, _ROWCH)], fbuf)
        pltpu.sync_copy(acc.at[pl.ds(r0, _ROWCH)], abuf)
        pltpu.sync_copy(deg.at[pl.ds(r0, _ROWCH)], dbuf)

        @pl.loop(0, _ROWCH)
        def _(i):
            inv = 1.0 / jnp.maximum(dbuf[i, :], 1.0)
            for q in range(dh // _LANES):
                sl = pl.ds(_LANES * q, _LANES)
                fbuf[i, sl] = fbuf[i, sl] + abuf[i, sl] * inv

        pltpu.sync_copy(fbuf, out2.at[pl.ds(coff + r0, _ROWCH)])


@jax.jit
def kernel(features, edge_index):
    n, d = features.shape
    e = edge_index.shape[1]
    dh = d // 2

    npad = _ceil_to(n + 1, _NS * _ROWCH)
    epad = _ceil_to(e, _NS * _IDXROWS * _BLK)

    src = edge_index[1].astype(jnp.int32)
    dst = edge_index[0].astype(jnp.int32)
    src_p = jnp.concatenate([src, jnp.zeros((epad - e,), jnp.int32)])
    dst_p = jnp.concatenate([dst, jnp.full((epad - e,), n, jnp.int32)])
    src2d = src_p.reshape(-1, _BLK)
    dst2d = dst_p.reshape(-1, _BLK)

    zrows = jnp.zeros((npad - n, dh), jnp.float32)
    feat2 = jnp.concatenate(
        [features[:, :dh], zrows, features[:, dh:], zrows], axis=0)

    ept_rows = src2d.shape[0] // _NS
    nfin = npad // (_NS * _ROWCH)

    mesh = plsc.VectorSubcoreMesh(core_axis_name="c", subcore_axis_name="s")
    body = functools.partial(_gnn_body, npad, ept_rows, nfin, dh)
    grid_kernel = pl.kernel(
        body,
        out_type=jax.ShapeDtypeStruct((_NC * npad, dh), jnp.float32),
        mesh=mesh,
        scratch_types=[
            pltpu.VMEM_SHARED((npad, dh), jnp.float32),
            pltpu.VMEM_SHARED((npad, _LANES), jnp.float32),
            pltpu.VMEM((_IDXROWS, _BLK), jnp.int32),
            pltpu.VMEM((_IDXROWS, _BLK), jnp.int32),
            pltpu.VMEM((_BLK, dh), jnp.float32),
            pltpu.VMEM((_BLK, _LANES), jnp.float32),
            pltpu.VMEM((_ROWCH, dh), jnp.float32),
            pltpu.VMEM((_ROWCH, dh), jnp.float32),
            pltpu.VMEM((_ROWCH, _LANES), jnp.float32),
        ],
    )
    out2 = grid_kernel(feat2, src2d, dst2d)
    return jnp.concatenate([out2[:n], out2[npad:npad + n]], axis=1)


# speedup vs baseline: 3.9946x; 3.9946x over previous
import functools

import jax
import jax.numpy as jnp
from jax import lax
from jax.experimental import pallas as pl
from jax.experimental.pallas import tpu as pltpu
from jax.experimental.pallas import tpu_sc as plsc

_NS = 16
_NC = 2
_LANES = 16
_BLK = 128
_IDXROWS = 8
_ROWCH = 128


def _ceil_to(x, m):
    return (x + m - 1) // m * m


def _gnn_body(npad, ept_rows, nfin, dh,
              feat2, src2d, dst2d, out2,
              acc, deg, sidx, didx, rows, ones_b, fbuf, abuf, dbuf):
    c = lax.axis_index("c")
    s = lax.axis_index("s")
    coff = c * npad

    one_v = jnp.full((_LANES,), 1.0, jnp.float32)
    zero_v = jnp.zeros((_LANES,), jnp.float32)

    @pl.loop(0, _BLK)
    def _(i):
        ones_b[i, :] = one_v
        dbuf[i, :] = zero_v
        for q in range(dh // _LANES):
            fbuf[i, pl.ds(_LANES * q, _LANES)] = zero_v

    @pl.loop(0, nfin)
    def _(k):
        r0 = (s * nfin + k) * _ROWCH
        pltpu.sync_copy(fbuf, acc.at[pl.ds(r0, _ROWCH)])
        pltpu.sync_copy(dbuf, deg.at[pl.ds(r0, _ROWCH)])

    plsc.subcore_barrier()

    ebase = s * ept_rows

    @pl.loop(0, ept_rows // _IDXROWS)
    def _(jb):
        rb = ebase + jb * _IDXROWS
        pltpu.sync_copy(src2d.at[pl.ds(rb, _IDXROWS)], sidx)
        pltpu.sync_copy(dst2d.at[pl.ds(rb, _IDXROWS)], didx)
        for r in range(_IDXROWS):
            for q in range(_BLK // _LANES):
                sl = pl.ds(_LANES * q, _LANES)
                sidx[r, sl] = sidx[r, sl] + coff
        for j in range(_IDXROWS):
            pltpu.sync_copy(feat2.at[sidx.at[j]], rows)
            pltpu.sync_copy(rows, acc.at[didx.at[j]], add=True)
            pltpu.sync_copy(ones_b, deg.at[didx.at[j]], add=True)

    plsc.subcore_barrier()

    @pl.loop(0, nfin)
    def _(k):
        r0 = (s * nfin + k) * _ROWCH
        pltpu.sync_copy(feat2.at[pl.ds(coff + r0, _ROWCH)], fbuf)
        pltpu.sync_copy(acc.at[pl.ds(r0, _ROWCH)], abuf)
        pltpu.sync_copy(deg.at[pl.ds(r0, _ROWCH)], dbuf)

        @pl.loop(0, _ROWCH)
        def _(i):
            inv = 1.0 / jnp.maximum(dbuf[i, :], 1.0)
            for q in range(dh // _LANES):
                sl = pl.ds(_LANES * q, _LANES)
                fbuf[i, sl] = fbuf[i, sl] + abuf[i, sl] * inv

        pltpu.sync_copy(fbuf, out2.at[pl.ds(coff + r0, _ROWCH)])


@jax.jit
def kernel(features, edge_index):
    n, d = features.shape
    e = edge_index.shape[1]
    dh = d // 2

    npad = _ceil_to(n + 1, _NS * _ROWCH)
    epad = _ceil_to(e, _NS * _IDXROWS * _BLK)

    src = edge_index[1].astype(jnp.int32)
    dst = edge_index[0].astype(jnp.int32)
    src_p = jnp.concatenate([src, jnp.zeros((epad - e,), jnp.int32)])
    dst_p = jnp.concatenate([dst, jnp.full((epad - e,), n, jnp.int32)])
    src2d = src_p.reshape(-1, _BLK)
    dst2d = dst_p.reshape(-1, _BLK)

    zrows = jnp.zeros((npad - n, dh), jnp.float32)
    feat2 = jnp.concatenate(
        [features[:, :dh], zrows, features[:, dh:], zrows], axis=0)

    ept_rows = src2d.shape[0] // _NS
    nfin = npad // (_NS * _ROWCH)

    mesh = plsc.VectorSubcoreMesh(core_axis_name="c", subcore_axis_name="s")
    body = functools.partial(_gnn_body, npad, ept_rows, nfin, dh)
    grid_kernel = pl.kernel(
        body,
        out_type=jax.ShapeDtypeStruct((_NC * npad, dh), jnp.float32),
        mesh=mesh,
        compiler_params=pltpu.CompilerParams(use_tc_tiling_on_sc=False),
        scratch_types=[
            pltpu.VMEM_SHARED((npad, dh), jnp.float32),
            pltpu.VMEM_SHARED((npad, _LANES), jnp.float32),
            pltpu.VMEM((_IDXROWS, _BLK), jnp.int32),
            pltpu.VMEM((_IDXROWS, _BLK), jnp.int32),
            pltpu.VMEM((_BLK, dh), jnp.float32),
            pltpu.VMEM((_BLK, _LANES), jnp.float32),
            pltpu.VMEM((_ROWCH, dh), jnp.float32),
            pltpu.VMEM((_ROWCH, dh), jnp.float32),
            pltpu.VMEM((_ROWCH, _LANES), jnp.float32),
        ],
    )
    out2 = grid_kernel(feat2, src2d, dst2d)
    return jnp.concatenate([out2[:n], out2[npad:npad + n]], axis=1)


# speedup vs baseline: 5.1948x; 1.3005x over previous
import functools

import jax
import jax.numpy as jnp
from jax import lax
from jax.experimental import pallas as pl
from jax.experimental.pallas import tpu as pltpu
from jax.experimental.pallas import tpu_sc as plsc

_NS = 16
_NC = 2
_LANES = 16
_BLK = 128
_IDXROWS = 8
_ROWCH = 128


def _ceil_to(x, m):
    return (x + m - 1) // m * m


_NSLOT = 4


def _gnn_body(npad, ept_rows, nfin, dh, nrows_total,
              feat2, src2d, dst2d, out2,
              acc, deg, sidx_a, didx_a, sidx_b, didx_b,
              rows, ones_b, degb,
              gsem, ssem, isem_a, isem_b):
    c = lax.axis_index("c")
    s = lax.axis_index("s")
    coff = c * npad

    fbuf, abuf = rows[0], rows[1]
    one_v = jnp.full((_LANES,), 1.0, jnp.float32)
    zero_v = jnp.zeros((_LANES,), jnp.float32)

    @pl.loop(0, _BLK)
    def _(i):
        ones_b[i, :] = zero_v
        for q in range(dh // _LANES):
            fbuf[i, pl.ds(_LANES * q, _LANES)] = zero_v

    @pl.loop(0, nfin)
    def _(k):
        r0 = (s * nfin + k) * _ROWCH
        pltpu.sync_copy(fbuf, acc.at[pl.ds(r0, _ROWCH)])
        pltpu.sync_copy(ones_b, deg.at[pl.ds(r0, _ROWCH)])

    @pl.loop(0, _BLK)
    def _(i):
        ones_b[i, :] = one_v

    plsc.subcore_barrier()

    ebase = s * ept_rows
    max_rb = nrows_total - _IDXROWS

    def prime(sidx, didx, isem, rb):
        pltpu.async_copy(src2d.at[pl.ds(rb, _IDXROWS)], sidx, isem)
        pltpu.async_copy(dst2d.at[pl.ds(rb, _IDXROWS)], didx, isem)

    def wait_idx(sidx, didx, isem):
        pltpu.make_async_copy(src2d.at[pl.ds(0, _IDXROWS)], sidx, isem).wait()
        pltpu.make_async_copy(dst2d.at[pl.ds(0, _IDXROWS)], didx, isem).wait()

    def drain_slot(t):
        pltpu.make_async_copy(rows[t], acc.at[didx_a.at[0]],
                              ssem.at[t]).wait()
        pltpu.make_async_copy(ones_b, deg.at[didx_a.at[0]],
                              ssem.at[t]).wait()

    def subround(sidx, didx, jbase, drain):
        gathers = []
        for t in range(_NSLOT):
            j = jbase + t
            if drain:
                drain_slot(t)
            for q in range(_BLK // _LANES):
                sl = pl.ds(_LANES * q, _LANES)
                sidx[j, sl] = sidx[j, sl] + coff
            gathers.append(pltpu.async_copy(
                feat2.at[sidx.at[j]], rows[t], gsem.at[t]))
            pltpu.async_copy(ones_b, deg.at[didx.at[j]], ssem.at[t],
                             add=True)
        for t in range(_NSLOT):
            gathers[t].wait()
            pltpu.async_copy(rows[t], acc.at[didx.at[jbase + t]], ssem.at[t],
                             add=True)

    def do_block(sidx, didx, isem, nsidx, ndidx, nisem, rb_next, first):
        wait_idx(sidx, didx, isem)
        if not first:
            for t in range(_NSLOT):
                drain_slot(t)
        prime(nsidx, ndidx, nisem, rb_next)
        subround(sidx, didx, 0, drain=False)
        subround(sidx, didx, _NSLOT, drain=True)

    prime(sidx_a, didx_a, isem_a, ebase)
    do_block(sidx_a, didx_a, isem_a, sidx_b, didx_b, isem_b,
             ebase + _IDXROWS, True)
    do_block(sidx_b, didx_b, isem_b, sidx_a, didx_a, isem_a,
             ebase + 2 * _IDXROWS, False)

    @pl.loop(1, ept_rows // (2 * _IDXROWS))
    def _(kb):
        rb = ebase + kb * 2 * _IDXROWS
        do_block(sidx_a, didx_a, isem_a, sidx_b, didx_b, isem_b,
                 rb + _IDXROWS, False)
        do_block(sidx_b, didx_b, isem_b, sidx_a, didx_a, isem_a,
                 jnp.minimum(rb + 2 * _IDXROWS, max_rb), False)

    for t in range(_NSLOT):
        drain_slot(t)
    wait_idx(sidx_a, didx_a, isem_a)

    plsc.subcore_barrier()

    @pl.loop(0, nfin)
    def _(k):
        r0 = (s * nfin + k) * _ROWCH
        pltpu.sync_copy(feat2.at[pl.ds(coff + r0, _ROWCH)], fbuf)
        pltpu.sync_copy(acc.at[pl.ds(r0, _ROWCH)], abuf)
        pltpu.sync_copy(deg.at[pl.ds(r0, _ROWCH)], degb)

        @pl.loop(0, _ROWCH)
        def _(i):
            inv = 1.0 / jnp.maximum(degb[i, :], 1.0)
            for q in range(dh // _LANES):
                sl = pl.ds(_LANES * q, _LANES)
                fbuf[i, sl] = fbuf[i, sl] + abuf[i, sl] * inv

        pltpu.sync_copy(fbuf, out2.at[pl.ds(coff + r0, _ROWCH)])


@jax.jit
def kernel(features, edge_index):
    n, d = features.shape
    e = edge_index.shape[1]
    dh = d // 2

    npad = _ceil_to(n + 1, _NS * _ROWCH)
    epad = _ceil_to(e, _NS * _IDXROWS * _BLK)

    src = edge_index[1].astype(jnp.int32)
    dst = edge_index[0].astype(jnp.int32)
    src_p = jnp.concatenate([src, jnp.zeros((epad - e,), jnp.int32)])
    dst_p = jnp.concatenate([dst, jnp.full((epad - e,), n, jnp.int32)])
    src2d = src_p.reshape(-1, _BLK)
    dst2d = dst_p.reshape(-1, _BLK)

    zrows = jnp.zeros((npad - n, dh), jnp.float32)
    feat2 = jnp.concatenate(
        [features[:, :dh], zrows, features[:, dh:], zrows], axis=0)

    ept_rows = src2d.shape[0] // _NS
    nfin = npad // (_NS * _ROWCH)

    mesh = plsc.VectorSubcoreMesh(core_axis_name="c", subcore_axis_name="s")
    body = functools.partial(_gnn_body, npad, ept_rows, nfin, dh,
                             src2d.shape[0])
    grid_kernel = pl.kernel(
        body,
        out_type=jax.ShapeDtypeStruct((_NC * npad, dh), jnp.float32),
        mesh=mesh,
        compiler_params=pltpu.CompilerParams(use_tc_tiling_on_sc=False),
        scratch_types=[
            pltpu.VMEM_SHARED((npad, dh), jnp.float32),
            pltpu.VMEM_SHARED((npad, _LANES), jnp.float32),
            pltpu.VMEM((_IDXROWS, _BLK), jnp.int32),
            pltpu.VMEM((_IDXROWS, _BLK), jnp.int32),
            pltpu.VMEM((_IDXROWS, _BLK), jnp.int32),
            pltpu.VMEM((_IDXROWS, _BLK), jnp.int32),
            [pltpu.VMEM((_BLK, dh), jnp.float32)
             for _ in range(_NSLOT)],
            pltpu.VMEM((_BLK, _LANES), jnp.float32),
            pltpu.VMEM((_ROWCH, _LANES), jnp.float32),
            pltpu.SemaphoreType.DMA((_NSLOT,)),
            pltpu.SemaphoreType.DMA((_NSLOT,)),
            pltpu.SemaphoreType.DMA,
            pltpu.SemaphoreType.DMA,
        ],
    )
    out2 = grid_kernel(feat2, src2d, dst2d)
    return jnp.concatenate([out2[:n], out2[npad:npad + n]], axis=1)


# speedup vs baseline: 5.4874x; 1.0563x over previous
import functools

import jax
import jax.numpy as jnp
from jax import lax
from jax.experimental import pallas as pl
from jax.experimental.pallas import tpu as pltpu
from jax.experimental.pallas import tpu_sc as plsc

_NS = 16
_NC = 2
_LANES = 16
_GEDGE = 256
_IDXROWS = 4
_RCH = 128


def _ceil_to(x, m):
    return (x + m - 1) // m * m


def _edge_body(npad, ept_rows, dh, nrows_total,
               feat2, src2d, dst2d, acc_out, deg_out,
               acc, deg, sidx_a, didx_a, sidx_b, didx_b,
               rows_p, rows_q, ones_b,
               gsem_p, gsem_q, ssem_p, ssem_q, isem_a, isem_b):
    c = lax.axis_index("c")
    s = lax.axis_index("s")
    coff = c * npad

    one_v = jnp.full((_LANES,), 1.0, jnp.float32)
    zero_v = jnp.zeros((_LANES,), jnp.float32)

    @pl.loop(0, _GEDGE)
    def _(i):
        ones_b[i, :] = zero_v

    @pl.loop(0, _RCH)
    def _(i):
        for q in range(dh // _LANES):
            rows_p[i, pl.ds(_LANES * q, _LANES)] = zero_v

    rpt = npad // _NS

    @pl.loop(0, rpt // _RCH)
    def _(k):
        r0 = s * rpt + k * _RCH
        pltpu.sync_copy(rows_p.at[pl.ds(0, _RCH)], acc.at[pl.ds(r0, _RCH)])
        pltpu.sync_copy(ones_b.at[pl.ds(0, _RCH)], deg.at[pl.ds(r0, _RCH)])

    @pl.loop(0, _GEDGE)
    def _(i):
        ones_b[i, :] = one_v

    plsc.subcore_barrier()

    ebase = s * ept_rows
    max_rb = nrows_total - _IDXROWS
    bufs = ((rows_p, gsem_p, ssem_p), (rows_q, gsem_q, ssem_q))

    def prime(sidx, didx, isem, rb):
        pltpu.async_copy(src2d.at[pl.ds(rb, _IDXROWS)], sidx, isem)
        pltpu.async_copy(dst2d.at[pl.ds(rb, _IDXROWS)], didx, isem)

    def wait_idx(sidx, didx, isem):
        pltpu.make_async_copy(src2d.at[pl.ds(0, _IDXROWS)], sidx, isem).wait()
        pltpu.make_async_copy(dst2d.at[pl.ds(0, _IDXROWS)], didx, isem).wait()

    def drain(buf, ssem):
        pltpu.make_async_copy(buf, acc.at[didx_a.at[0]], ssem).wait()
        pltpu.make_async_copy(ones_b, deg.at[didx_a.at[0]], ssem).wait()

    def do_round(sidx, didx, j0, prev_descs):
        gathers, descs = [], []
        for t, (buf, gsem, ssem) in enumerate(bufs):
            j = j0 + t
            if prev_descs is not None:
                for d_ in prev_descs[t]:
                    d_.wait()
            gathers.append(pltpu.async_copy(feat2.at[sidx.at[j]], buf, gsem))
            dg = pltpu.async_copy(ones_b, deg.at[didx.at[j]], ssem, add=True)
            descs.append([dg])
        for t, (buf, gsem, ssem) in enumerate(bufs):
            gathers[t].wait()
            descs[t].append(pltpu.async_copy(
                buf, acc.at[didx.at[j0 + t]], ssem, add=True))
        return descs

    def do_block(sidx, didx, isem, nsidx, ndidx, nisem, rb_next, first):
        wait_idx(sidx, didx, isem)
        if not first:
            drain(rows_p, ssem_p)
            drain(rows_q, ssem_q)
        prime(nsidx, ndidx, nisem, rb_next)
        for j in range(_IDXROWS):
            for q in range(_GEDGE // _LANES):
                sl = pl.ds(_LANES * q, _LANES)
                sidx[j, sl] = sidx[j, sl] + coff
        descs = do_round(sidx, didx, 0, None)
        do_round(sidx, didx, 2, descs)

    prime(sidx_a, didx_a, isem_a, ebase)
    do_block(sidx_a, didx_a, isem_a, sidx_b, didx_b, isem_b,
             ebase + _IDXROWS, True)
    do_block(sidx_b, didx_b, isem_b, sidx_a, didx_a, isem_a,
             ebase + 2 * _IDXROWS, False)

    @pl.loop(1, ept_rows // (2 * _IDXROWS))
    def _(kb):
        rb = ebase + kb * 2 * _IDXROWS
        do_block(sidx_a, didx_a, isem_a, sidx_b, didx_b, isem_b,
                 rb + _IDXROWS, False)
        do_block(sidx_b, didx_b, isem_b, sidx_a, didx_a, isem_a,
                 jnp.minimum(rb + 2 * _IDXROWS, max_rb), False)

    drain(rows_p, ssem_p)
    drain(rows_q, ssem_q)
    wait_idx(sidx_a, didx_a, isem_a)

    plsc.subcore_barrier()

    @pl.loop(0, rpt // _RCH)
    def _(k):
        r0 = s * rpt + k * _RCH
        pltpu.sync_copy(acc.at[pl.ds(r0, _RCH)], rows_p.at[pl.ds(0, _RCH)])
        pltpu.sync_copy(rows_p.at[pl.ds(0, _RCH)],
                        acc_out.at[c, pl.ds(r0, _RCH)])
        pltpu.sync_copy(deg.at[pl.ds(r0, _RCH)], ones_b.at[pl.ds(0, _RCH)])
        pltpu.sync_copy(ones_b.at[pl.ds(0, _RCH)],
                        deg_out.at[c, pl.ds(r0, _RCH)])


def _combine_body(feat_ref, a0_ref, a1_ref, d0_ref, d1_ref, out_ref):
    inv0 = 1.0 / jnp.maximum(d0_ref[0][:, 0:1], 1.0)
    inv1 = 1.0 / jnp.maximum(d1_ref[0][:, 0:1], 1.0)
    agg = jnp.concatenate([a0_ref[0] * inv0, a1_ref[0] * inv1], axis=1)
    out_ref[...] = feat_ref[...] + agg


@jax.jit
def kernel(features, edge_index):
    n, d = features.shape
    e = edge_index.shape[1]
    dh = d // 2

    npad = _ceil_to(n + 1, _NS * _RCH)
    epad = _ceil_to(e, _NS * 2 * _IDXROWS * _GEDGE)

    src2d = jnp.pad(edge_index[1].astype(jnp.int32),
                    (0, epad - e)).reshape(-1, _GEDGE)
    dst2d = jnp.pad(edge_index[0].astype(jnp.int32), (0, epad - e),
                    constant_values=n).reshape(-1, _GEDGE)

    zrows = jnp.zeros((npad - n, dh), jnp.float32)
    feat2 = jnp.concatenate(
        [features[:, :dh], zrows, features[:, dh:], zrows], axis=0)

    ept_rows = src2d.shape[0] // _NS

    mesh = plsc.VectorSubcoreMesh(core_axis_name="c", subcore_axis_name="s")
    body = functools.partial(_edge_body, npad, ept_rows, dh, src2d.shape[0])
    edge_kernel = pl.kernel(
        body,
        out_type=[jax.ShapeDtypeStruct((_NC, npad, dh), jnp.float32),
                  jax.ShapeDtypeStruct((_NC, npad, _LANES), jnp.float32)],
        mesh=mesh,
        compiler_params=pltpu.CompilerParams(use_tc_tiling_on_sc=False),
        scratch_types=[
            pltpu.VMEM_SHARED((npad, dh), jnp.float32),
            pltpu.VMEM_SHARED((npad, _LANES), jnp.float32),
            pltpu.VMEM((_IDXROWS, _GEDGE), jnp.int32),
            pltpu.VMEM((_IDXROWS, _GEDGE), jnp.int32),
            pltpu.VMEM((_IDXROWS, _GEDGE), jnp.int32),
            pltpu.VMEM((_IDXROWS, _GEDGE), jnp.int32),
            pltpu.VMEM((_GEDGE, dh), jnp.float32),
            pltpu.VMEM((_GEDGE, dh), jnp.float32),
            pltpu.VMEM((_GEDGE, _LANES), jnp.float32),
            pltpu.SemaphoreType.DMA,
            pltpu.SemaphoreType.DMA,
            pltpu.SemaphoreType.DMA,
            pltpu.SemaphoreType.DMA,
            pltpu.SemaphoreType.DMA,
            pltpu.SemaphoreType.DMA,
        ],
    )
    acc2, deg2 = edge_kernel(feat2, src2d, dst2d)

    blk = 2000
    out = pl.pallas_call(
        _combine_body,
        grid=(n // blk,),
        in_specs=[
            pl.BlockSpec((blk, d), lambda i: (i, 0)),
            pl.BlockSpec((1, blk, dh), lambda i: (0, i, 0)),
            pl.BlockSpec((1, blk, dh), lambda i: (1, i, 0)),
            pl.BlockSpec((1, blk, _LANES), lambda i: (0, i, 0)),
            pl.BlockSpec((1, blk, _LANES), lambda i: (1, i, 0)),
        ],
        out_specs=pl.BlockSpec((blk, d), lambda i: (i, 0)),
        out_shape=jax.ShapeDtypeStruct((n, d), jnp.float32),
    )(features, acc2, acc2, deg2, deg2)
    return out


# speedup vs baseline: 5.8409x; 1.0644x over previous
import functools

import jax
import jax.numpy as jnp
from jax import lax
from jax.experimental import pallas as pl
from jax.experimental.pallas import tpu as pltpu
from jax.experimental.pallas import tpu_sc as plsc

_NS = 16
_NC = 2
_LANES = 16
_GEDGE = 256
_IDXROWS = 4
_RCH = 128


def _ceil_to(x, m):
    return (x + m - 1) // m * m


def _edge_body(npad, ept_rows, dh, nrows_total,
               feat2, src2d, dst2d, acc_out, deg_out,
               acc, deg, sidx_a, didx_a, sidx_b, didx_b,
               rows_p, rows_q, ones_b,
               gsem_p, gsem_q, ssem_p, ssem_q, isem_a, isem_b):
    c = lax.axis_index("c")
    s = lax.axis_index("s")
    coff = c * npad

    one_v = jnp.full((_LANES,), 1.0, jnp.float32)
    zero_v = jnp.zeros((_LANES,), jnp.float32)

    @pl.loop(0, _GEDGE)
    def _(i):
        ones_b[i, :] = zero_v

    @pl.loop(0, _RCH)
    def _(i):
        for q in range(dh // _LANES):
            rows_p[i, pl.ds(_LANES * q, _LANES)] = zero_v

    rpt = npad // _NS

    @pl.loop(0, rpt // _RCH)
    def _(k):
        r0 = s * rpt + k * _RCH
        pltpu.sync_copy(rows_p.at[pl.ds(0, _RCH)], acc.at[pl.ds(r0, _RCH)])
        pltpu.sync_copy(ones_b.at[pl.ds(0, _RCH)], deg.at[pl.ds(r0, _RCH)])

    @pl.loop(0, _GEDGE)
    def _(i):
        ones_b[i, :] = one_v

    plsc.subcore_barrier()

    ebase = s * ept_rows
    max_rb = nrows_total - _IDXROWS
    bufs = ((rows_p, gsem_p, ssem_p), (rows_q, gsem_q, ssem_q))

    def prime(sidx, didx, isem, rb):
        pltpu.async_copy(src2d.at[pl.ds(rb, _IDXROWS)], sidx, isem)
        pltpu.async_copy(dst2d.at[pl.ds(rb, _IDXROWS)], didx, isem)

    def wait_idx(sidx, didx, isem):
        pltpu.make_async_copy(src2d.at[pl.ds(0, _IDXROWS)], sidx, isem).wait()
        pltpu.make_async_copy(dst2d.at[pl.ds(0, _IDXROWS)], didx, isem).wait()

    def drain(buf, ssem):
        pltpu.make_async_copy(buf, acc.at[didx_a.at[0]], ssem).wait()
        pltpu.make_async_copy(ones_b, deg.at[didx_a.at[0]], ssem).wait()

    def drain_nodeg(buf, ssem):
        pltpu.make_async_copy(buf, acc.at[didx_a.at[0]], ssem).wait()

    def do_round(sidx, didx, j0, prev_descs, with_deg):
        gathers, descs = [], []
        for t, (buf, gsem, ssem) in enumerate(bufs):
            j = j0 + t
            if prev_descs is not None:
                for d_ in prev_descs[t]:
                    d_.wait()
            gathers.append(pltpu.async_copy(feat2.at[sidx.at[j]], buf, gsem))
            if with_deg:
                dg = pltpu.async_copy(ones_b, deg.at[didx.at[j]], ssem,
                                      add=True)
                descs.append([dg])
            else:
                descs.append([])
        for t, (buf, gsem, ssem) in enumerate(bufs):
            gathers[t].wait()
            descs[t].append(pltpu.async_copy(
                buf, acc.at[didx.at[j0 + t]], ssem, add=True))
        return descs

    def do_block(sidx, didx, isem, nsidx, ndidx, nisem, rb_next, first,
                 with_deg):
        wait_idx(sidx, didx, isem)
        dr = drain if with_deg else drain_nodeg
        if not first:
            dr(rows_p, ssem_p)
            dr(rows_q, ssem_q)
        prime(nsidx, ndidx, nisem, rb_next)
        for j in range(_IDXROWS):
            for q in range(_GEDGE // _LANES):
                sl = pl.ds(_LANES * q, _LANES)
                sidx[j, sl] = sidx[j, sl] + coff
        descs = do_round(sidx, didx, 0, None, with_deg)
        do_round(sidx, didx, 2, descs, with_deg)

    def edge_pass(with_deg):
        dr = drain if with_deg else drain_nodeg
        prime(sidx_a, didx_a, isem_a, ebase)
        do_block(sidx_a, didx_a, isem_a, sidx_b, didx_b, isem_b,
                 ebase + _IDXROWS, True, with_deg)
        do_block(sidx_b, didx_b, isem_b, sidx_a, didx_a, isem_a,
                 ebase + 2 * _IDXROWS, False, with_deg)

        @pl.loop(1, ept_rows // (2 * _IDXROWS))
        def _(kb):
            rb = ebase + kb * 2 * _IDXROWS
            do_block(sidx_a, didx_a, isem_a, sidx_b, didx_b, isem_b,
                     rb + _IDXROWS, False, with_deg)
            do_block(sidx_b, didx_b, isem_b, sidx_a, didx_a, isem_a,
                     jnp.minimum(rb + 2 * _IDXROWS, max_rb), False, with_deg)

        dr(rows_p, ssem_p)
        dr(rows_q, ssem_q)
        wait_idx(sidx_a, didx_a, isem_a)

    deg_duty = jnp.logical_xor(c == 1, s < _NS // 2)

    @pl.when(deg_duty)
    def _():
        edge_pass(True)

    @pl.when(jnp.logical_not(deg_duty))
    def _():
        edge_pass(False)

    plsc.subcore_barrier()

    @pl.loop(0, rpt // _RCH)
    def _(k):
        r0 = s * rpt + k * _RCH
        pltpu.sync_copy(acc.at[pl.ds(r0, _RCH)], rows_p.at[pl.ds(0, _RCH)])
        pltpu.sync_copy(rows_p.at[pl.ds(0, _RCH)],
                        acc_out.at[c, pl.ds(r0, _RCH)])
        pltpu.sync_copy(deg.at[pl.ds(r0, _RCH)], ones_b.at[pl.ds(0, _RCH)])
        pltpu.sync_copy(ones_b.at[pl.ds(0, _RCH)],
                        deg_out.at[c, pl.ds(r0, _RCH)])


def _combine_body(feat_ref, a0_ref, a1_ref, d0_ref, d1_ref, out_ref):
    dtot = d0_ref[0][:, 0:1] + d1_ref[0][:, 0:1]
    inv = 1.0 / jnp.maximum(dtot, 1.0)
    agg = jnp.concatenate([a0_ref[0] * inv, a1_ref[0] * inv], axis=1)
    out_ref[...] = feat_ref[...] + agg


@jax.jit
def kernel(features, edge_index):
    n, d = features.shape
    e = edge_index.shape[1]
    dh = d // 2

    npad = _ceil_to(n + 1, _NS * _RCH)
    epad = _ceil_to(e, _NS * 2 * _IDXROWS * _GEDGE)

    ei = jnp.pad(edge_index.astype(jnp.int32), ((0, 0), (0, epad - e)),
                 constant_values=n)
    src2d = ei[1].reshape(-1, _GEDGE)
    dst2d = ei[0].reshape(-1, _GEDGE)

    zrows = jnp.zeros((npad - n, dh), jnp.float32)
    feat2 = jnp.concatenate(
        [features[:, :dh], zrows, features[:, dh:], zrows], axis=0)

    ept_rows = src2d.shape[0] // _NS

    mesh = plsc.VectorSubcoreMesh(core_axis_name="c", subcore_axis_name="s")
    body = functools.partial(_edge_body, npad, ept_rows, dh, src2d.shape[0])
    edge_kernel = pl.kernel(
        body,
        out_type=[jax.ShapeDtypeStruct((_NC, npad, dh), jnp.float32),
                  jax.ShapeDtypeStruct((_NC, npad, _LANES), jnp.float32)],
        mesh=mesh,
        compiler_params=pltpu.CompilerParams(use_tc_tiling_on_sc=False),
        scratch_types=[
            pltpu.VMEM_SHARED((npad, dh), jnp.float32),
            pltpu.VMEM_SHARED((npad, _LANES), jnp.float32),
            pltpu.VMEM((_IDXROWS, _GEDGE), jnp.int32),
            pltpu.VMEM((_IDXROWS, _GEDGE), jnp.int32),
            pltpu.VMEM((_IDXROWS, _GEDGE), jnp.int32),
            pltpu.VMEM((_IDXROWS, _GEDGE), jnp.int32),
            pltpu.VMEM((_GEDGE, dh), jnp.float32),
            pltpu.VMEM((_GEDGE, dh), jnp.float32),
            pltpu.VMEM((_GEDGE, _LANES), jnp.float32),
            pltpu.SemaphoreType.DMA,
            pltpu.SemaphoreType.DMA,
            pltpu.SemaphoreType.DMA,
            pltpu.SemaphoreType.DMA,
            pltpu.SemaphoreType.DMA,
            pltpu.SemaphoreType.DMA,
        ],
    )
    acc2, deg2 = edge_kernel(feat2, src2d, dst2d)

    blk = 2000
    out = pl.pallas_call(
        _combine_body,
        grid=(n // blk,),
        in_specs=[
            pl.BlockSpec((blk, d), lambda i: (i, 0)),
            pl.BlockSpec((1, blk, dh), lambda i: (0, i, 0)),
            pl.BlockSpec((1, blk, dh), lambda i: (1, i, 0)),
            pl.BlockSpec((1, blk, _LANES), lambda i: (0, i, 0)),
            pl.BlockSpec((1, blk, _LANES), lambda i: (1, i, 0)),
        ],
        out_specs=pl.BlockSpec((blk, d), lambda i: (i, 0)),
        out_shape=jax.ShapeDtypeStruct((n, d), jnp.float32),
    )(features, acc2, acc2, deg2, deg2)
    return out
